# Initial kernel scaffold; baseline (speedup 1.0000x reference)
import functools

import jax
import jax.numpy as jnp
from jax import lax
from jax.experimental import pallas as pl
from jax.experimental.pallas import tpu as pltpu
from jax.experimental.pallas import tpu_sc as plsc

N = 10000
E = 320000
NF = 128
NH = 256

NC = 2
NS = 16
CHUNK = 128
N_PAD = 10240
STRIPE = N_PAD // NS
E_PAD = 323584
EPT = E_PAD // (NC * NS)
NCHUNKS = EPT // CHUNK

_mesh = plsc.VectorSubcoreMesh(core_axis_name="c", subcore_axis_name="s")


@functools.partial(
    pl.kernel,
    mesh=_mesh,
    out_type=jax.ShapeDtypeStruct((NC * 2 * N_PAD, 16), jnp.float32),
    scratch_types=[
        pltpu.VMEM((CHUNK,), jnp.int32),
        pltpu.VMEM((CHUNK,), jnp.int32),
        pltpu.VMEM((CHUNK, 16), jnp.float32),
        pltpu.VMEM_SHARED((N_PAD, 16), jnp.float32),
        pltpu.VMEM_SHARED((N_PAD, 16), jnp.float32),
    ],
)
def _degrees_sc(src_hbm, dst_hbm, z16_hbm, ones_hbm, out_hbm,
                isrc_v, idst_v, ones_v, acc_out, acc_in):
    c = lax.axis_index("c")
    s = lax.axis_index("s")
    pltpu.sync_copy(ones_hbm, ones_v)
    pltpu.sync_copy(z16_hbm, acc_out.at[pl.ds(s * STRIPE, STRIPE)])
    pltpu.sync_copy(z16_hbm, acc_in.at[pl.ds(s * STRIPE, STRIPE)])
    plsc.subcore_barrier()
    base = (c * NS + s) * EPT

    @pl.loop(0, NCHUNKS)
    def _(k):
        off = base + k * CHUNK
        pltpu.sync_copy(src_hbm.at[pl.ds(off, CHUNK)], isrc_v)
        pltpu.sync_copy(dst_hbm.at[pl.ds(off, CHUNK)], idst_v)
        pltpu.sync_copy(ones_v, acc_out.at[isrc_v], add=True)
        pltpu.sync_copy(ones_v, acc_in.at[idst_v], add=True)

    plsc.subcore_barrier()
    pltpu.sync_copy(acc_out.at[pl.ds(s * STRIPE, STRIPE)],
                    out_hbm.at[pl.ds((c * 2) * N_PAD + s * STRIPE, STRIPE)])
    pltpu.sync_copy(acc_in.at[pl.ds(s * STRIPE, STRIPE)],
                    out_hbm.at[pl.ds((c * 2 + 1) * N_PAD + s * STRIPE, STRIPE)])


@functools.partial(
    pl.kernel,
    mesh=_mesh,
    out_type=jax.ShapeDtypeStruct((NC * N_PAD, NF), jnp.float32),
    scratch_types=[
        pltpu.VMEM((CHUNK,), jnp.int32),
        pltpu.VMEM((CHUNK,), jnp.int32),
        pltpu.VMEM((CHUNK, NF), jnp.float32),
        pltpu.VMEM_SHARED((N_PAD, NF), jnp.float32),
        pltpu.SemaphoreType.DMA,
    ],
)
def _aggregate_sc(table_hbm, src_hbm, dst_hbm, zrow_hbm, out_hbm,
                  isrc_v, idst_v, rows_v, acc, sem):
    c = lax.axis_index("c")
    s = lax.axis_index("s")
    pltpu.sync_copy(zrow_hbm, acc.at[pl.ds(s * STRIPE, STRIPE)])
    plsc.subcore_barrier()
    base = (c * NS + s) * EPT

    @pl.loop(0, NCHUNKS)
    def _(k):
        off = base + k * CHUNK
        pltpu.sync_copy(src_hbm.at[pl.ds(off, CHUNK)], isrc_v)
        pltpu.sync_copy(dst_hbm.at[pl.ds(off, CHUNK)], idst_v)
        pltpu.async_copy(table_hbm.at[isrc_v], rows_v, sem).wait()
        pltpu.sync_copy(rows_v, acc.at[idst_v], add=True)

    plsc.subcore_barrier()
    pltpu.sync_copy(acc.at[pl.ds(s * STRIPE, STRIPE)],
                    out_hbm.at[pl.ds(c * N_PAD + s * STRIPE, STRIPE)])


_R = 256
_GRID = N_PAD // _R


def _scale_body(feat_ref, outr_ref, o_ref):
    o_ref[...] = feat_ref[...] * outr_ref[...][:, None]


def _mm_body(p0_ref, p1_ref, inr_ref, outr_ref, w1_ref, b1_ref, w2_ref,
             o0_ref, o1_ref):
    a = (p0_ref[0] + p1_ref[0]) * inr_ref[...][:, None]
    h = jnp.dot(a, w1_ref[...], preferred_element_type=jnp.float32)
    h = jnp.maximum(h + b1_ref[...], 0.0) * outr_ref[...][:, None]
    g = jnp.dot(h, w2_ref[...], preferred_element_type=jnp.float32)
    o0_ref[...] = g[:, :NF]
    o1_ref[...] = g[:, NF:]


def _final_body(q00_ref, q01_ref, q10_ref, q11_ref, inr_ref, b2_ref, w3_ref,
                o_ref):
    inr = inr_ref[...][:, None]
    m0 = (q00_ref[0] + q01_ref[0]) * inr + b2_ref[...][:, :NF]
    m1 = (q10_ref[0] + q11_ref[0]) * inr + b2_ref[...][:, NF:]
    h0 = jnp.maximum(m0, 0.0)
    h1 = jnp.maximum(m1, 0.0)
    o_ref[...] = (jnp.dot(h0, w3_ref[...][:NF], preferred_element_type=jnp.float32)
                  + jnp.dot(h1, w3_ref[...][NF:], preferred_element_type=jnp.float32))


def _vec_spec():
    return pl.BlockSpec((_R,), lambda i: (i,))


def _row_spec():
    return pl.BlockSpec((_R, NF), lambda i: (i, 0))


def _part_spec(core):
    return pl.BlockSpec((1, _R, NF), lambda i, c=core: (c, i, 0))


def _full(shape):
    return pl.BlockSpec(shape, lambda i: tuple(0 for _ in shape))


def _scale_tc(feat_p, outr):
    return pl.pallas_call(
        _scale_body,
        grid=(_GRID,),
        in_specs=[_row_spec(), _vec_spec()],
        out_specs=_row_spec(),
        out_shape=jax.ShapeDtypeStruct((N_PAD, NF), jnp.float32),
    )(feat_p, outr)


def _mm_tc(parts, inr, outr, W1, b1r, W2):
    return pl.pallas_call(
        _mm_body,
        grid=(_GRID,),
        in_specs=[_part_spec(0), _part_spec(1), _vec_spec(), _vec_spec(),
                  _full((NF, 2 * NH)), _full((1, 2 * NH)), _full((2 * NH, NH))],
        out_specs=(_row_spec(), _row_spec()),
        out_shape=(jax.ShapeDtypeStruct((N_PAD, NF), jnp.float32),
                   jax.ShapeDtypeStruct((N_PAD, NF), jnp.float32)),
    )(parts, parts, inr, outr, W1, b1r, W2)


def _final_tc(q0, q1, inr, b2r, W3p):
    return pl.pallas_call(
        _final_body,
        grid=(_GRID,),
        in_specs=[_part_spec(0), _part_spec(1), _part_spec(0), _part_spec(1),
                  _vec_spec(), _full((1, NH)), _full((NH, NF))],
        out_specs=_row_spec(),
        out_shape=jax.ShapeDtypeStruct((N_PAD, NF), jnp.float32),
    )(q0, q0, q1, q1, inr, b2r, W3p)


def kernel(feat, edge_index, W1, b1, W2, b2, W3, b3):
    src = edge_index[0].astype(jnp.int32)
    dst = edge_index[1].astype(jnp.int32)
    padi = jnp.full((E_PAD - E,), N, jnp.int32)
    src_p = jnp.concatenate([src, padi])
    dst_p = jnp.concatenate([dst, padi])

    z16 = jnp.zeros((STRIPE, 16), jnp.float32)
    ones16 = jnp.ones((CHUNK, 16), jnp.float32)
    zrow = jnp.zeros((STRIPE, NF), jnp.float32)

    degs = _degrees_sc(src_p, dst_p, z16, ones16).reshape(NC, 2, N_PAD, 16)
    outr = jnp.clip(degs[0, 0, :, 0] + degs[1, 0, :, 0], 1.0, None) ** -0.5
    inr = jnp.clip(degs[0, 1, :, 0] + degs[1, 1, :, 0], 1.0, None) ** -0.5

    feat_p = jnp.concatenate([feat, jnp.zeros((N_PAD - N, NF), jnp.float32)])
    xs = _scale_tc(feat_p, outr)

    p1 = _aggregate_sc(xs, src_p, dst_p, zrow).reshape(NC, N_PAD, NF)
    g2a, g2b = _mm_tc(p1, inr, outr, W1, b1.reshape(1, -1), W2)

    q0 = _aggregate_sc(g2a, src_p, dst_p, zrow).reshape(NC, N_PAD, NF)
    q1 = _aggregate_sc(g2b, src_p, dst_p, zrow).reshape(NC, N_PAD, NF)

    W3p = jnp.concatenate([W3, jnp.zeros((NH, NF - 1), jnp.float32)], axis=1)
    outp = _final_tc(q0, q1, inr, b2.reshape(1, -1), W3p)
    return outp[:N, 0:1] + b3


# speedup vs baseline: 4.0372x; 4.0372x over previous
import functools

import jax
import jax.numpy as jnp
from jax import lax
from jax.experimental import pallas as pl
from jax.experimental.pallas import tpu as pltpu
from jax.experimental.pallas import tpu_sc as plsc

N = 10000
E = 320000
NF = 128
NH = 256

NC = 2
NS = 16
CHUNK = 128
N_PAD = 10240
STRIPE = N_PAD // NS
E_PAD = 323584
EPT = E_PAD // (NC * NS)
NCHUNKS = EPT // CHUNK

_mesh = plsc.VectorSubcoreMesh(core_axis_name="c", subcore_axis_name="s")


@functools.partial(
    pl.kernel,
    mesh=_mesh,
    out_type=jax.ShapeDtypeStruct((NC * N_PAD, NF), jnp.float32),
    scratch_types=[
        pltpu.VMEM((CHUNK,), jnp.int32),
        pltpu.VMEM((CHUNK, NF), jnp.float32),
        pltpu.VMEM_SHARED((N_PAD, NF), jnp.float32),
    ],
)
def _count_sc(idx_hbm, zrow_hbm, ones_hbm, out_hbm, idx_v, ones_v, acc):
    c = lax.axis_index("c")
    s = lax.axis_index("s")
    pltpu.sync_copy(ones_hbm, ones_v)
    pltpu.sync_copy(zrow_hbm, acc.at[pl.ds(s * STRIPE, STRIPE)])
    plsc.subcore_barrier()
    base = (c * NS + s) * EPT

    @pl.loop(0, NCHUNKS)
    def _(k):
        off = base + k * CHUNK
        pltpu.sync_copy(idx_hbm.at[pl.ds(off, CHUNK)], idx_v)
        pltpu.sync_copy(ones_v, acc.at[idx_v], add=True)

    plsc.subcore_barrier()
    pltpu.sync_copy(acc.at[pl.ds(s * STRIPE, STRIPE)],
                    out_hbm.at[pl.ds(c * N_PAD + s * STRIPE, STRIPE)])


@functools.partial(
    pl.kernel,
    mesh=_mesh,
    out_type=jax.ShapeDtypeStruct((NC * N_PAD, NF), jnp.float32),
    scratch_types=[
        pltpu.VMEM((CHUNK,), jnp.int32),
        pltpu.VMEM((CHUNK,), jnp.int32),
        pltpu.VMEM((CHUNK, NF), jnp.float32),
        pltpu.VMEM_SHARED((N_PAD, NF), jnp.float32),
        pltpu.SemaphoreType.DMA,
    ],
)
def _aggregate_sc(table_hbm, src_hbm, dst_hbm, zrow_hbm, out_hbm,
                  isrc_v, idst_v, rows_v, acc, sem):
    c = lax.axis_index("c")
    s = lax.axis_index("s")
    pltpu.sync_copy(zrow_hbm, acc.at[pl.ds(s * STRIPE, STRIPE)])
    plsc.subcore_barrier()
    base = (c * NS + s) * EPT

    @pl.loop(0, NCHUNKS)
    def _(k):
        off = base + k * CHUNK
        pltpu.sync_copy(src_hbm.at[pl.ds(off, CHUNK)], isrc_v)
        pltpu.sync_copy(dst_hbm.at[pl.ds(off, CHUNK)], idst_v)
        pltpu.async_copy(table_hbm.at[isrc_v], rows_v, sem).wait()
        pltpu.sync_copy(rows_v, acc.at[idst_v], add=True)

    plsc.subcore_barrier()
    pltpu.sync_copy(acc.at[pl.ds(s * STRIPE, STRIPE)],
                    out_hbm.at[pl.ds(c * N_PAD + s * STRIPE, STRIPE)])


_R = 256
_GRID = N_PAD // _R


def _scale_body(feat_ref, outr_ref, o_ref):
    o_ref[...] = feat_ref[...] * outr_ref[...][:, None]


def _mm_body(p0_ref, p1_ref, inr_ref, outr_ref, w1_ref, b1_ref, w2_ref,
             o0_ref, o1_ref):
    a = (p0_ref[0] + p1_ref[0]) * inr_ref[...][:, None]
    h = jnp.dot(a, w1_ref[...], preferred_element_type=jnp.float32)
    h = jnp.maximum(h + b1_ref[...], 0.0) * outr_ref[...][:, None]
    g = jnp.dot(h, w2_ref[...], preferred_element_type=jnp.float32)
    o0_ref[...] = g[:, :NF]
    o1_ref[...] = g[:, NF:]


def _final_body(q00_ref, q01_ref, q10_ref, q11_ref, inr_ref, b2_ref, w3_ref,
                o_ref):
    inr = inr_ref[...][:, None]
    m0 = (q00_ref[0] + q01_ref[0]) * inr + b2_ref[...][:, :NF]
    m1 = (q10_ref[0] + q11_ref[0]) * inr + b2_ref[...][:, NF:]
    h0 = jnp.maximum(m0, 0.0)
    h1 = jnp.maximum(m1, 0.0)
    o_ref[...] = (jnp.dot(h0, w3_ref[...][:NF], preferred_element_type=jnp.float32)
                  + jnp.dot(h1, w3_ref[...][NF:], preferred_element_type=jnp.float32))


def _vec_spec():
    return pl.BlockSpec((_R,), lambda i: (i,))


def _row_spec():
    return pl.BlockSpec((_R, NF), lambda i: (i, 0))


def _part_spec(core):
    return pl.BlockSpec((1, _R, NF), lambda i, c=core: (c, i, 0))


def _full(shape):
    return pl.BlockSpec(shape, lambda i: tuple(0 for _ in shape))


def _scale_tc(feat_p, outr):
    return pl.pallas_call(
        _scale_body,
        grid=(_GRID,),
        in_specs=[_row_spec(), _vec_spec()],
        out_specs=_row_spec(),
        out_shape=jax.ShapeDtypeStruct((N_PAD, NF), jnp.float32),
    )(feat_p, outr)


def _mm_tc(parts, inr, outr, W1, b1r, W2):
    return pl.pallas_call(
        _mm_body,
        grid=(_GRID,),
        in_specs=[_part_spec(0), _part_spec(1), _vec_spec(), _vec_spec(),
                  _full((NF, 2 * NH)), _full((1, 2 * NH)), _full((2 * NH, NH))],
        out_specs=(_row_spec(), _row_spec()),
        out_shape=(jax.ShapeDtypeStruct((N_PAD, NF), jnp.float32),
                   jax.ShapeDtypeStruct((N_PAD, NF), jnp.float32)),
    )(parts, parts, inr, outr, W1, b1r, W2)


def _final_tc(q0, q1, inr, b2r, W3p):
    return pl.pallas_call(
        _final_body,
        grid=(_GRID,),
        in_specs=[_part_spec(0), _part_spec(1), _part_spec(0), _part_spec(1),
                  _vec_spec(), _full((1, NH)), _full((NH, NF))],
        out_specs=_row_spec(),
        out_shape=jax.ShapeDtypeStruct((N_PAD, NF), jnp.float32),
    )(q0, q0, q1, q1, inr, b2r, W3p)


def kernel(feat, edge_index, W1, b1, W2, b2, W3, b3):
    src = edge_index[0].astype(jnp.int32)
    dst = edge_index[1].astype(jnp.int32)
    padi = jnp.full((E_PAD - E,), N, jnp.int32)
    src_p = jnp.concatenate([src, padi])
    dst_p = jnp.concatenate([dst, padi])

    ones_row = jnp.ones((CHUNK, NF), jnp.float32)
    zrow = jnp.zeros((STRIPE, NF), jnp.float32)

    cnt_src = _count_sc(src_p, zrow, ones_row).reshape(NC, N_PAD, NF)
    cnt_dst = _count_sc(dst_p, zrow, ones_row).reshape(NC, N_PAD, NF)
    outr = jnp.clip(cnt_src[0, :, 0] + cnt_src[1, :, 0], 1.0, None) ** -0.5
    inr = jnp.clip(cnt_dst[0, :, 0] + cnt_dst[1, :, 0], 1.0, None) ** -0.5

    feat_p = jnp.concatenate([feat, jnp.zeros((N_PAD - N, NF), jnp.float32)])
    xs = _scale_tc(feat_p, outr)

    p1 = _aggregate_sc(xs, src_p, dst_p, zrow).reshape(NC, N_PAD, NF)
    g2a, g2b = _mm_tc(p1, inr, outr, W1, b1.reshape(1, -1), W2)

    q0 = _aggregate_sc(g2a, src_p, dst_p, zrow).reshape(NC, N_PAD, NF)
    q1 = _aggregate_sc(g2b, src_p, dst_p, zrow).reshape(NC, N_PAD, NF)

    W3p = jnp.concatenate([W3, jnp.zeros((NH, NF - 1), jnp.float32)], axis=1)
    outp = _final_tc(q0, q1, inr, b2.reshape(1, -1), W3p)
    return outp[:N, 0:1] + b3
